# baseline (speedup 1.0000x reference)
import jax
import jax.numpy as jnp
from jax.experimental import pallas as pl
from jax.experimental.pallas import tpu as pltpu

THR = 0.9
_BLOCK = 2000


def _body(nidx_ref, score_ref, out_ref):
    n = nidx_ref[...]
    s = score_ref[...]
    ones = jnp.ones((n.shape[0], 1), dtype=jnp.float32)
    full = jnp.concatenate([ones, s], axis=1)
    out_ref[...] = jnp.where(full >= THR, n, -1)


def kernel(nidx, score, specweights, tidxs):
    V, K = nidx.shape
    score2 = score.reshape(V, K - 1)
    grid = V // _BLOCK
    return pl.pallas_call(
        _body,
        grid=(grid,),
        in_specs=[
            pl.BlockSpec((_BLOCK, K), lambda i: (i, 0)),
            pl.BlockSpec((_BLOCK, K - 1), lambda i: (i, 0)),
        ],
        out_specs=pl.BlockSpec((_BLOCK, K), lambda i: (i, 0)),
        out_shape=jax.ShapeDtypeStruct((V, K), jnp.int32),
        compiler_params=pltpu.CompilerParams(
            dimension_semantics=("arbitrary",),
        ),
    )(nidx, score2)


# speedup vs baseline: 1.1664x; 1.1664x over previous
import jax
import jax.numpy as jnp
from jax.experimental import pallas as pl
from jax.experimental.pallas import tpu as pltpu

THR = 0.9
_BLOCK = 2048


def _body(nidx_ref, scoret_ref, out_ref):
    n = nidx_ref[...]
    st = scoret_ref[...]
    s = jnp.transpose(st.reshape(st.shape[0], st.shape[2]))
    ones = jnp.ones((n.shape[0], 1), dtype=jnp.float32)
    full = jnp.concatenate([ones, s], axis=1)
    out_ref[...] = jnp.where(full >= THR, n, -1)


def kernel(nidx, score, specweights, tidxs):
    V, K = nidx.shape
    score_t = jnp.transpose(score, (1, 2, 0))
    grid = (V + _BLOCK - 1) // _BLOCK
    return pl.pallas_call(
        _body,
        grid=(grid,),
        in_specs=[
            pl.BlockSpec((_BLOCK, K), lambda i: (i, 0)),
            pl.BlockSpec((K - 1, 1, _BLOCK), lambda i: (0, 0, i)),
        ],
        out_specs=pl.BlockSpec((_BLOCK, K), lambda i: (i, 0)),
        out_shape=jax.ShapeDtypeStruct((V, K), jnp.int32),
        compiler_params=pltpu.CompilerParams(
            dimension_semantics=("arbitrary",),
        ),
    )(nidx, score_t)
